# baseline (speedup 1.0000x reference)
import jax
import jax.numpy as jnp
from jax import lax
from jax.experimental import pallas as pl
from jax.experimental.pallas import tpu as pltpu
from jax.experimental.pallas import tpu_sc as plsc

_G, _N, _F = 8, 512, 128
_H = 128
_D_TP = 64
_N_MLP = 256
_N_GCN = 4
_B = 64
_GPB = 2
_STEPS = _G // _GPB
_L = 16
_NCHUNK = _N // _L


def _softmax(z):
    z = z - jnp.max(z, axis=-1, keepdims=True)
    e = jnp.exp(z)
    return e / jnp.sum(e, axis=-1, keepdims=True)


def _gcn_body(gx_ref, ga_ref, W_in_ref, b_in_ref, W_gcn_ref, b_gcn_ref,
              xf_ref):
    def bdot(a, b):
        return jnp.dot(a.astype(jnp.bfloat16), b.astype(jnp.bfloat16),
                       preferred_element_type=jnp.float32)

    for j in range(_GPB):
        gx = gx_ref[j]
        ga = ga_ref[j].astype(jnp.bfloat16)
        x = bdot(gx, W_in_ref[...])
        x = jnp.maximum(x + b_in_ref[...], 0.0)
        to_add = x
        for i in range(_N_GCN):
            z = jnp.dot(ga, x.astype(jnp.bfloat16),
                        preferred_element_type=jnp.float32)
            z = bdot(z, W_gcn_ref[i])
            z = z + b_gcn_ref[i]
            if i < _N_GCN - 1:
                x = jnp.maximum(z, 0.0)
            else:
                x = _softmax(z)
        xf_ref[j] = x + to_add


def _run_gcn(graph_xs_all, graph_as_all, W_in, b_in, W_gcn, b_gcn):
    full = lambda shape: pl.BlockSpec(shape, lambda g: (0,) * len(shape))
    return pl.pallas_call(
        _gcn_body,
        grid=(_STEPS,),
        in_specs=[
            pl.BlockSpec((_GPB, _N, _F), lambda g: (g, 0, 0)),
            pl.BlockSpec((_GPB, _N, _N), lambda g: (g, 0, 0)),
            full((_F, _H)), full((_H,)),
            full((_N_GCN, _H, _H)), full((_N_GCN, _H)),
        ],
        out_specs=pl.BlockSpec((_GPB, _N, _H), lambda g: (g, 0, 0)),
        out_shape=jax.ShapeDtypeStruct((_G, _N, _H), jnp.float32),
    )(graph_xs_all, graph_as_all, W_in, b_in, W_gcn, b_gcn)


_MAIN = 128
_MCH = _MAIN // _L


def _pool_body(xf_hbm, mask_hbm, idx_hbm, cov_hbm,
               idx_v, mask_v, idxbuf0, idxbuf1, rows0, rows1, rows_t, cov_v,
               sem0, sem1, semt):
    info = plsc.get_sparse_core_info()
    nc = info.num_cores
    wid = lax.axis_index("s") * nc + lax.axis_index("c")
    b0 = wid * 2

    pltpu.sync_copy(idx_hbm, idx_v)
    pltpu.sync_copy(mask_hbm.at[pl.ds(b0, 2)], mask_v)
    lanes = lax.broadcasted_iota(jnp.int32, (_L,), 0)

    idxbufs = (idxbuf0, idxbuf1)
    rows = (rows0, rows1)
    sems = (sem0, sem1)
    cnts, handles = [], []

    for j in range(2):
        ibuf = idxbufs[j]
        g_vec = plsc.load_gather(idx_v, [jnp.full((_L,), b0 + j, jnp.int32)])
        base_vec = g_vec * _N

        for k in range(_NCHUNK + 2):
            ibuf[pl.ds(k * _L, _L)] = base_vec

        def compact(c, cnt, ibuf=ibuf, base_vec=base_vec, j=j):
            mv = mask_v[j, pl.ds(c * _L, _L)]
            msk = mv > 0.0
            mi = jnp.where(msk, 1, 0).astype(jnp.int32)
            pos = jnp.cumsum(mi) - 1
            glob = base_vec + c * _L + lanes
            plsc.store_scatter(ibuf, [pos + cnt], glob, mask=msk)
            return cnt + jnp.sum(mi)

        cnt = lax.fori_loop(0, _NCHUNK, compact, jnp.int32(0))
        cnts.append(cnt)
        handles.append(
            pltpu.async_copy(xf_hbm.at[ibuf.at[pl.ds(0, _MAIN)]],
                             rows[j], sems[j]))

    for j in range(2):
        cnt = cnts[j]
        handles[j].wait()
        rv = rows[j]

        acc = []
        for s in range(_H // _L):
            partials = []
            for c in range(_MCH):
                p = rv[c * _L, pl.ds(s * _L, _L)]
                for r in range(1, _L):
                    p = p + rv[c * _L + r, pl.ds(s * _L, _L)]
                partials.append(p)
            while len(partials) > 1:
                partials = [a + b for a, b in
                            zip(partials[::2], partials[1::2])]
            acc.append(partials[0])

        cmain = jnp.minimum(cnt, _MAIN)
        padf = jnp.full((_L,), (_MAIN - cmain).astype(jnp.float32))
        acc = [acc[s] - padf * rv[0, pl.ds(s * _L, _L)]
               for s in range(_H // _L)]

        nch = (cnt + _L - 1) // _L

        def tail_step(t, acc_t, ibuf=idxbufs[j], cnt=cnt):
            iv = ibuf[pl.ds(t * _L, _L)]
            pltpu.async_copy(xf_hbm.at[iv], rows_t, semt).wait()
            out = []
            for s in range(_H // _L):
                seg = jnp.zeros((_L,), jnp.float32)
                for r in range(_L):
                    valid = (t * _L + r) < cnt
                    row = rows_t[r, pl.ds(s * _L, _L)]
                    seg = seg + jnp.where(valid, row, 0.0)
                out.append(acc_t[s] + seg)
            return tuple(out)

        acc = list(lax.fori_loop(_MCH, nch, tail_step, tuple(acc)))

        denom_vec = jnp.maximum(jnp.full((_L,), cnt.astype(jnp.float32)), 1.0)
        scale = 1.0 / denom_vec
        for s in range(_H // _L):
            cov_v[j, pl.ds(s * _L, _L)] = acc[s] * scale

    pltpu.sync_copy(cov_v, cov_hbm.at[pl.ds(b0, 2)])


def _run_pool(xf_flat, mask_f, idx):
    mesh = plsc.VectorSubcoreMesh(core_axis_name="c", subcore_axis_name="s")
    return pl.kernel(
        _pool_body,
        out_type=jax.ShapeDtypeStruct((_B, _H), jnp.float32),
        mesh=mesh,
        compiler_params=pltpu.CompilerParams(needs_layout_passes=False),
        scratch_types=[
            pltpu.VMEM((_B,), jnp.int32),
            pltpu.VMEM((2, _N), jnp.float32),
            pltpu.VMEM((_N + 2 * _L,), jnp.int32),
            pltpu.VMEM((_N + 2 * _L,), jnp.int32),
            pltpu.VMEM((_MAIN, _H), jnp.float32),
            pltpu.VMEM((_MAIN, _H), jnp.float32),
            pltpu.VMEM((_L, _H), jnp.float32),
            pltpu.VMEM((2, _H), jnp.float32),
            pltpu.SemaphoreType.DMA,
            pltpu.SemaphoreType.DMA,
            pltpu.SemaphoreType.DMA,
        ],
    )(xf_flat, mask_f, idx)


def _head_body(cov_ref, tp_ref, W_tp1_ref, b_tp1_ref, W_tp2_ref, b_tp2_ref,
               W_f1_ref, b_f1_ref, W_f2_ref, b_f2_ref, out_ref):
    t = jnp.dot(tp_ref[...], W_tp1_ref[...],
                preferred_element_type=jnp.float32) + b_tp1_ref[...]
    t = jnp.maximum(t, 0.0)
    t = jnp.dot(t, W_tp2_ref[...],
                preferred_element_type=jnp.float32) + b_tp2_ref[...]
    tp_e = _softmax(t)
    h = (jnp.dot(cov_ref[...], W_f1_ref[:_H],
                 preferred_element_type=jnp.float32)
         + jnp.dot(tp_e, W_f1_ref[_H:],
                   preferred_element_type=jnp.float32)
         + b_f1_ref[...])
    h = jnp.maximum(h, 0.0)
    o = jnp.dot(h, W_f2_ref[...],
                preferred_element_type=jnp.float32) + b_f2_ref[...]
    out_ref[...] = 1.0 / (1.0 + jnp.exp(-o))


def _run_head(cov, test_parameters, W_tp1, b_tp1, W_tp2, b_tp2,
              W_f1, b_f1, W_f2, b_f2):
    return pl.pallas_call(
        _head_body,
        out_shape=jax.ShapeDtypeStruct((_B, 1), jnp.float32),
    )(cov, test_parameters, W_tp1, b_tp1, W_tp2, b_tp2,
      W_f1, b_f1, W_f2, b_f2)


def kernel(test_parameters, graph, coverpoint_mask, graph_xs_all, graph_as_all,
           W_in, b_in, W_gcn, b_gcn, W_tp1, b_tp1, W_tp2, b_tp2,
           W_f1, b_f1, W_f2, b_f2):
    idx = graph[:, 0].astype(jnp.int32)
    mask_f = coverpoint_mask.astype(jnp.float32)

    xf = _run_gcn(graph_xs_all, graph_as_all, W_in, b_in, W_gcn, b_gcn)
    cov = _run_pool(xf.reshape(_G * _N, _H), mask_f, idx)
    out = _run_head(cov, test_parameters, W_tp1, b_tp1, W_tp2, b_tp2,
                    W_f1, b_f1, W_f2, b_f2)
    return out


# speedup vs baseline: 1.1118x; 1.1118x over previous
import jax
import jax.numpy as jnp
from jax import lax
from jax.experimental import pallas as pl
from jax.experimental.pallas import tpu as pltpu
from jax.experimental.pallas import tpu_sc as plsc

_G, _N, _F = 8, 512, 128
_H = 128
_D_TP = 64
_N_MLP = 256
_N_GCN = 4
_B = 64
_GPB = 2
_STEPS = _G // _GPB
_L = 16
_NCHUNK = _N // _L


def _softmax(z):
    z = z - jnp.max(z, axis=-1, keepdims=True)
    e = jnp.exp(z)
    return e / jnp.sum(e, axis=-1, keepdims=True)


def _gcn_body(gx_ref, ga_ref, W_in_ref, b_in_ref, W_gcn_ref, b_gcn_ref,
              xf_ref):
    def bdot(a, b):
        return jnp.dot(a.astype(jnp.bfloat16), b.astype(jnp.bfloat16),
                       preferred_element_type=jnp.float32)

    for j in range(_GPB):
        gx = gx_ref[j]
        ga = ga_ref[j].astype(jnp.bfloat16)
        x = bdot(gx, W_in_ref[...])
        x = jnp.maximum(x + b_in_ref[...], 0.0)
        to_add = x
        for i in range(_N_GCN):
            z = jnp.dot(ga, x.astype(jnp.bfloat16),
                        preferred_element_type=jnp.float32)
            z = bdot(z, W_gcn_ref[i])
            z = z + b_gcn_ref[i]
            if i < _N_GCN - 1:
                x = jnp.maximum(z, 0.0)
            else:
                x = _softmax(z)
        xf_ref[j] = x + to_add


def _run_gcn(graph_xs_all, graph_as_all, W_in, b_in, W_gcn, b_gcn):
    full = lambda shape: pl.BlockSpec(shape, lambda g: (0,) * len(shape))
    return pl.pallas_call(
        _gcn_body,
        grid=(_STEPS,),
        in_specs=[
            pl.BlockSpec((_GPB, _N, _F), lambda g: (g, 0, 0)),
            pl.BlockSpec((_GPB, _N, _N), lambda g: (g, 0, 0)),
            full((_F, _H)), full((_H,)),
            full((_N_GCN, _H, _H)), full((_N_GCN, _H)),
        ],
        out_specs=pl.BlockSpec((_GPB, _N, _H), lambda g: (g, 0, 0)),
        out_shape=jax.ShapeDtypeStruct((_G, _N, _H), jnp.float32),
    )(graph_xs_all, graph_as_all, W_in, b_in, W_gcn, b_gcn)


_MAIN = 128
_MCH = _MAIN // _L


def _pool_body(xf_hbm, mask_hbm, idx_hbm, cov_hbm,
               idx_v, mask_v, idxbuf0, idxbuf1, rows0, rows1, rows_t, cov_v,
               sem0, sem1, semt):
    info = plsc.get_sparse_core_info()
    nc = info.num_cores
    wid = lax.axis_index("s") * nc + lax.axis_index("c")
    b0 = wid * 2

    pltpu.sync_copy(idx_hbm, idx_v)
    pltpu.sync_copy(mask_hbm.at[pl.ds(b0, 2)], mask_v)
    lanes = lax.broadcasted_iota(jnp.int32, (_L,), 0)

    idxbufs = (idxbuf0, idxbuf1)
    rows = (rows0, rows1)
    sems = (sem0, sem1)
    cnts, handles = [], []

    for j in range(2):
        ibuf = idxbufs[j]
        g_vec = plsc.load_gather(idx_v, [jnp.full((_L,), b0 + j, jnp.int32)])
        base_vec = g_vec * _N

        for k in range(_NCHUNK + 2):
            ibuf[pl.ds(k * _L, _L)] = base_vec

        def compact(c, cnt, ibuf=ibuf, base_vec=base_vec, j=j):
            mv = mask_v[j, pl.ds(c * _L, _L)]
            msk = mv > 0.0
            mi = jnp.where(msk, 1, 0).astype(jnp.int32)
            pos = jnp.cumsum(mi) - 1
            glob = base_vec + c * _L + lanes
            plsc.store_scatter(ibuf, [pos + cnt], glob, mask=msk)
            return cnt + jnp.sum(mi)

        cnt = lax.fori_loop(0, _NCHUNK, compact, jnp.int32(0))
        cnts.append(cnt)
        handles.append(
            pltpu.async_copy(xf_hbm.at[ibuf.at[pl.ds(0, _MAIN)]],
                             rows[j], sems[j]))

    for j in range(2):
        cnt = cnts[j]
        handles[j].wait()
        rv = rows[j]

        def acc_step(c, acc_c, rv=rv):
            out = []
            for s in range(_H // _L):
                seg = rv[c * _L, pl.ds(s * _L, _L)]
                for r in range(1, _L):
                    seg = seg + rv[c * _L + r, pl.ds(s * _L, _L)]
                out.append(acc_c[s] + seg)
            return tuple(out)

        acc0 = tuple(jnp.zeros((_L,), jnp.float32) for _ in range(_H // _L))
        acc = list(lax.fori_loop(0, _MCH, acc_step, acc0))

        cmain = jnp.minimum(cnt, _MAIN)
        padf = jnp.full((_L,), (_MAIN - cmain).astype(jnp.float32))
        acc = [acc[si] - padf * rv[0, pl.ds(si * _L, _L)]
               for si in range(_H // _L)]

        nch = (cnt + _L - 1) // _L

        def tail_step(t, acc_t, ibuf=idxbufs[j], cnt=cnt):
            iv = ibuf[pl.ds(t * _L, _L)]
            pltpu.async_copy(xf_hbm.at[iv], rows_t, semt).wait()
            out = []
            for s in range(_H // _L):
                seg = jnp.zeros((_L,), jnp.float32)
                for r in range(_L):
                    valid = (t * _L + r) < cnt
                    row = rows_t[r, pl.ds(s * _L, _L)]
                    seg = seg + jnp.where(valid, row, 0.0)
                out.append(acc_t[s] + seg)
            return tuple(out)

        acc = list(lax.fori_loop(_MCH, nch, tail_step, tuple(acc)))

        denom_vec = jnp.maximum(jnp.full((_L,), cnt.astype(jnp.float32)), 1.0)
        scale = 1.0 / denom_vec
        for s in range(_H // _L):
            cov_v[j, pl.ds(s * _L, _L)] = acc[s] * scale

    pltpu.sync_copy(cov_v, cov_hbm.at[pl.ds(b0, 2)])


def _run_pool(xf_flat, mask_f, idx):
    mesh = plsc.VectorSubcoreMesh(core_axis_name="c", subcore_axis_name="s")
    return pl.kernel(
        _pool_body,
        out_type=jax.ShapeDtypeStruct((_B, _H), jnp.float32),
        mesh=mesh,
        compiler_params=pltpu.CompilerParams(needs_layout_passes=False),
        scratch_types=[
            pltpu.VMEM((_B,), jnp.int32),
            pltpu.VMEM((2, _N), jnp.float32),
            pltpu.VMEM((_N + 2 * _L,), jnp.int32),
            pltpu.VMEM((_N + 2 * _L,), jnp.int32),
            pltpu.VMEM((_MAIN, _H), jnp.float32),
            pltpu.VMEM((_MAIN, _H), jnp.float32),
            pltpu.VMEM((_L, _H), jnp.float32),
            pltpu.VMEM((2, _H), jnp.float32),
            pltpu.SemaphoreType.DMA,
            pltpu.SemaphoreType.DMA,
            pltpu.SemaphoreType.DMA,
        ],
    )(xf_flat, mask_f, idx)


def _head_body(cov_ref, tp_ref, W_tp1_ref, b_tp1_ref, W_tp2_ref, b_tp2_ref,
               W_f1_ref, b_f1_ref, W_f2_ref, b_f2_ref, out_ref):
    t = jnp.dot(tp_ref[...], W_tp1_ref[...],
                preferred_element_type=jnp.float32) + b_tp1_ref[...]
    t = jnp.maximum(t, 0.0)
    t = jnp.dot(t, W_tp2_ref[...],
                preferred_element_type=jnp.float32) + b_tp2_ref[...]
    tp_e = _softmax(t)
    h = (jnp.dot(cov_ref[...], W_f1_ref[:_H],
                 preferred_element_type=jnp.float32)
         + jnp.dot(tp_e, W_f1_ref[_H:],
                   preferred_element_type=jnp.float32)
         + b_f1_ref[...])
    h = jnp.maximum(h, 0.0)
    o = jnp.dot(h, W_f2_ref[...],
                preferred_element_type=jnp.float32) + b_f2_ref[...]
    out_ref[...] = 1.0 / (1.0 + jnp.exp(-o))


def _run_head(cov, test_parameters, W_tp1, b_tp1, W_tp2, b_tp2,
              W_f1, b_f1, W_f2, b_f2):
    return pl.pallas_call(
        _head_body,
        out_shape=jax.ShapeDtypeStruct((_B, 1), jnp.float32),
    )(cov, test_parameters, W_tp1, b_tp1, W_tp2, b_tp2,
      W_f1, b_f1, W_f2, b_f2)


def kernel(test_parameters, graph, coverpoint_mask, graph_xs_all, graph_as_all,
           W_in, b_in, W_gcn, b_gcn, W_tp1, b_tp1, W_tp2, b_tp2,
           W_f1, b_f1, W_f2, b_f2):
    idx = graph[:, 0].astype(jnp.int32)
    mask_f = coverpoint_mask.astype(jnp.float32)

    xf = _run_gcn(graph_xs_all, graph_as_all, W_in, b_in, W_gcn, b_gcn)
    cov = _run_pool(xf.reshape(_G * _N, _H), mask_f, idx)
    out = _run_head(cov, test_parameters, W_tp1, b_tp1, W_tp2, b_tp2,
                    W_f1, b_f1, W_f2, b_f2)
    return out


# speedup vs baseline: 1.9214x; 1.7283x over previous
import jax
import jax.numpy as jnp
from jax import lax
from jax.experimental import pallas as pl
from jax.experimental.pallas import tpu as pltpu
from jax.experimental.pallas import tpu_sc as plsc

_G, _N, _F = 8, 512, 128
_H = 128
_D_TP = 64
_N_MLP = 256
_N_GCN = 4
_B = 64
_GPB = 2
_STEPS = _G // _GPB
_L = 16
_NCHUNK = _N // _L


def _softmax(z):
    z = z - jnp.max(z, axis=-1, keepdims=True)
    e = jnp.exp(z)
    return e / jnp.sum(e, axis=-1, keepdims=True)


def _gcn_body(gx_ref, ga_ref, W_in_ref, b_in_ref, W_gcn_ref, b_gcn_ref,
              xf_ref):
    def bdot(a, b):
        return jnp.dot(a.astype(jnp.bfloat16), b.astype(jnp.bfloat16),
                       preferred_element_type=jnp.float32)

    for j in range(_GPB):
        gx = gx_ref[j]
        ga = ga_ref[j].astype(jnp.bfloat16)
        x = bdot(gx, W_in_ref[...])
        x = jnp.maximum(x + b_in_ref[...], 0.0)
        to_add = x
        for i in range(_N_GCN):
            z = jnp.dot(ga, x.astype(jnp.bfloat16),
                        preferred_element_type=jnp.float32)
            z = bdot(z, W_gcn_ref[i])
            z = z + b_gcn_ref[i]
            if i < _N_GCN - 1:
                x = jnp.maximum(z, 0.0)
            else:
                x = _softmax(z)
        xf_ref[j] = x + to_add


def _run_gcn(graph_xs_all, graph_as_all, W_in, b_in, W_gcn, b_gcn):
    full = lambda shape: pl.BlockSpec(shape, lambda g: (0,) * len(shape))
    return pl.pallas_call(
        _gcn_body,
        grid=(_STEPS,),
        in_specs=[
            pl.BlockSpec((_GPB, _N, _F), lambda g: (g, 0, 0)),
            pl.BlockSpec((_GPB, _N, _N), lambda g: (g, 0, 0)),
            full((_F, _H)), full((_H,)),
            full((_N_GCN, _H, _H)), full((_N_GCN, _H)),
        ],
        out_specs=pl.BlockSpec((_GPB, _N, _H), lambda g: (g, 0, 0)),
        out_shape=jax.ShapeDtypeStruct((_G, _N, _H), jnp.float32),
    )(graph_xs_all, graph_as_all, W_in, b_in, W_gcn, b_gcn)


_MAIN = 128
_MCH = _MAIN // _L


def _pool_body(xf_hbm, mask_hbm, idx_hbm, cov_hbm,
               idx_v, mask_v, idxbuf0, idxbuf1, rows0, rows1, rows_t, cov_v,
               sem0, sem1, semt):
    info = plsc.get_sparse_core_info()
    nc = info.num_cores
    wid = lax.axis_index("s") * nc + lax.axis_index("c")
    b0 = wid * 2

    pltpu.sync_copy(idx_hbm, idx_v)
    pltpu.sync_copy(mask_hbm.at[pl.ds(b0, 2)], mask_v)
    lanes = lax.broadcasted_iota(jnp.int32, (_L,), 0)

    idxbufs = (idxbuf0, idxbuf1)
    rows = (rows0, rows1)
    sems = (sem0, sem1)
    cnts, handles = [], []

    for j in range(2):
        ibuf = idxbufs[j]
        g_vec = plsc.load_gather(idx_v, [jnp.full((_L,), b0 + j, jnp.int32)])
        base_vec = g_vec * _N

        for k in range(_NCHUNK + 2):
            ibuf[pl.ds(k * _L, _L)] = base_vec

        def compact(c, cnt, ibuf=ibuf, base_vec=base_vec, j=j):
            mv = mask_v[j, pl.ds(c * _L, _L)]
            msk = mv > 0.0
            mi = jnp.where(msk, 1, 0).astype(jnp.int32)
            pos = jnp.cumsum(mi) - 1
            glob = base_vec + c * _L + lanes
            plsc.store_scatter(ibuf, [pos + cnt], glob, mask=msk)
            return cnt + jnp.sum(mi)

        cnt = lax.fori_loop(0, _NCHUNK, compact, jnp.int32(0))
        cnts.append(cnt)
        nchm = jnp.minimum((cnt + _L - 1) // _L, _MCH)

        def fire(t, carry, ibuf=ibuf, rv=rows[j], sem=sems[j]):
            iv = ibuf[pl.ds(t * _L, _L)]
            pltpu.async_copy(xf_hbm.at[iv], rv.at[pl.ds(t * _L, _L)], sem)
            return carry
        lax.fori_loop(0, nchm, fire, jnp.int32(0))

    for j in range(2):
        cnt = cnts[j]
        nchm = jnp.minimum((cnt + _L - 1) // _L, _MCH)
        rv = rows[j]

        def drain(t, carry, ibuf=idxbufs[j], rv=rv, sem=sems[j]):
            iv = ibuf[pl.ds(0, _L)]
            pltpu.make_async_copy(xf_hbm.at[iv],
                                  rv.at[pl.ds(0, _L)], sem).wait()
            return carry
        lax.fori_loop(0, nchm, drain, jnp.int32(0))

        def acc_step(c, acc_c, rv=rv, cnt=cnt):
            out = []
            for s in range(_H // _L):
                seg = acc_c[s]
                for r in range(_L):
                    valid = (c * _L + r) < cnt
                    row = rv[c * _L + r, pl.ds(s * _L, _L)]
                    seg = seg + jnp.where(valid, row, 0.0)
                out.append(seg)
            return tuple(out)

        acc0 = tuple(jnp.zeros((_L,), jnp.float32) for _ in range(_H // _L))
        acc = list(lax.fori_loop(0, nchm, acc_step, acc0))

        nch = (cnt + _L - 1) // _L

        def tail_step(t, acc_t, ibuf=idxbufs[j], cnt=cnt):
            iv = ibuf[pl.ds(t * _L, _L)]
            pltpu.async_copy(xf_hbm.at[iv], rows_t, semt).wait()
            out = []
            for s in range(_H // _L):
                seg = jnp.zeros((_L,), jnp.float32)
                for r in range(_L):
                    valid = (t * _L + r) < cnt
                    row = rows_t[r, pl.ds(s * _L, _L)]
                    seg = seg + jnp.where(valid, row, 0.0)
                out.append(acc_t[s] + seg)
            return tuple(out)

        acc = list(lax.fori_loop(_MCH, nch, tail_step, tuple(acc)))

        denom_vec = jnp.maximum(jnp.full((_L,), cnt.astype(jnp.float32)), 1.0)
        scale = 1.0 / denom_vec
        for s in range(_H // _L):
            cov_v[j, pl.ds(s * _L, _L)] = acc[s] * scale

    pltpu.sync_copy(cov_v, cov_hbm.at[pl.ds(b0, 2)])


def _run_pool(xf_flat, mask_f, idx):
    mesh = plsc.VectorSubcoreMesh(core_axis_name="c", subcore_axis_name="s")
    return pl.kernel(
        _pool_body,
        out_type=jax.ShapeDtypeStruct((_B, _H), jnp.float32),
        mesh=mesh,
        compiler_params=pltpu.CompilerParams(needs_layout_passes=False),
        scratch_types=[
            pltpu.VMEM((_B,), jnp.int32),
            pltpu.VMEM((2, _N), jnp.float32),
            pltpu.VMEM((_N + 2 * _L,), jnp.int32),
            pltpu.VMEM((_N + 2 * _L,), jnp.int32),
            pltpu.VMEM((_MAIN, _H), jnp.float32),
            pltpu.VMEM((_MAIN, _H), jnp.float32),
            pltpu.VMEM((_L, _H), jnp.float32),
            pltpu.VMEM((2, _H), jnp.float32),
            pltpu.SemaphoreType.DMA,
            pltpu.SemaphoreType.DMA,
            pltpu.SemaphoreType.DMA,
        ],
    )(xf_flat, mask_f, idx)


def _head_body(cov_ref, tp_ref, W_tp1_ref, b_tp1_ref, W_tp2_ref, b_tp2_ref,
               W_f1_ref, b_f1_ref, W_f2_ref, b_f2_ref, out_ref):
    t = jnp.dot(tp_ref[...], W_tp1_ref[...],
                preferred_element_type=jnp.float32) + b_tp1_ref[...]
    t = jnp.maximum(t, 0.0)
    t = jnp.dot(t, W_tp2_ref[...],
                preferred_element_type=jnp.float32) + b_tp2_ref[...]
    tp_e = _softmax(t)
    h = (jnp.dot(cov_ref[...], W_f1_ref[:_H],
                 preferred_element_type=jnp.float32)
         + jnp.dot(tp_e, W_f1_ref[_H:],
                   preferred_element_type=jnp.float32)
         + b_f1_ref[...])
    h = jnp.maximum(h, 0.0)
    o = jnp.dot(h, W_f2_ref[...],
                preferred_element_type=jnp.float32) + b_f2_ref[...]
    out_ref[...] = 1.0 / (1.0 + jnp.exp(-o))


def _run_head(cov, test_parameters, W_tp1, b_tp1, W_tp2, b_tp2,
              W_f1, b_f1, W_f2, b_f2):
    return pl.pallas_call(
        _head_body,
        out_shape=jax.ShapeDtypeStruct((_B, 1), jnp.float32),
    )(cov, test_parameters, W_tp1, b_tp1, W_tp2, b_tp2,
      W_f1, b_f1, W_f2, b_f2)


def kernel(test_parameters, graph, coverpoint_mask, graph_xs_all, graph_as_all,
           W_in, b_in, W_gcn, b_gcn, W_tp1, b_tp1, W_tp2, b_tp2,
           W_f1, b_f1, W_f2, b_f2):
    idx = graph[:, 0].astype(jnp.int32)
    mask_f = coverpoint_mask.astype(jnp.float32)

    xf = _run_gcn(graph_xs_all, graph_as_all, W_in, b_in, W_gcn, b_gcn)
    cov = _run_pool(xf.reshape(_G * _N, _H), mask_f, idx)
    out = _run_head(cov, test_parameters, W_tp1, b_tp1, W_tp2, b_tp2,
                    W_f1, b_f1, W_f2, b_f2)
    return out


# speedup vs baseline: 1.9233x; 1.0010x over previous
import jax
import jax.numpy as jnp
from jax import lax
from jax.experimental import pallas as pl
from jax.experimental.pallas import tpu as pltpu
from jax.experimental.pallas import tpu_sc as plsc

_G, _N, _F = 8, 512, 128
_H = 128
_D_TP = 64
_N_MLP = 256
_N_GCN = 4
_B = 64
_GPB = 2
_STEPS = _G // _GPB
_L = 16
_NCHUNK = _N // _L


def _softmax(z):
    z = z - jnp.max(z, axis=-1, keepdims=True)
    e = jnp.exp(z)
    return e / jnp.sum(e, axis=-1, keepdims=True)


def _gcn_body(gx_ref, ga_ref, W_in_ref, b_in_ref, W_gcn_ref, b_gcn_ref,
              xf_ref):
    def bdot(a, b):
        return jnp.dot(a.astype(jnp.bfloat16), b.astype(jnp.bfloat16),
                       preferred_element_type=jnp.float32)

    for j in range(_GPB):
        gx = gx_ref[j]
        ga = ga_ref[j].astype(jnp.bfloat16)
        x = bdot(gx, W_in_ref[...])
        x = jnp.maximum(x + b_in_ref[...], 0.0)
        to_add = x
        for i in range(_N_GCN):
            z = jnp.dot(ga, x.astype(jnp.bfloat16),
                        preferred_element_type=jnp.float32)
            z = bdot(z, W_gcn_ref[i])
            z = z + b_gcn_ref[i]
            if i < _N_GCN - 1:
                x = jnp.maximum(z, 0.0)
            else:
                x = _softmax(z)
        xf_ref[j] = x + to_add


def _run_gcn(graph_xs_all, graph_as_all, W_in, b_in, W_gcn, b_gcn):
    full = lambda shape: pl.BlockSpec(shape, lambda g: (0,) * len(shape))
    return pl.pallas_call(
        _gcn_body,
        grid=(_STEPS,),
        in_specs=[
            pl.BlockSpec((_GPB, _N, _F), lambda g: (g, 0, 0)),
            pl.BlockSpec((_GPB, _N, _N), lambda g: (g, 0, 0)),
            full((_F, _H)), full((_H,)),
            full((_N_GCN, _H, _H)), full((_N_GCN, _H)),
        ],
        out_specs=pl.BlockSpec((_GPB, _N, _H), lambda g: (g, 0, 0)),
        out_shape=jax.ShapeDtypeStruct((_G, _N, _H), jnp.float32),
    )(graph_xs_all, graph_as_all, W_in, b_in, W_gcn, b_gcn)


_MAIN = 128
_MCH = _MAIN // _L


def _pool_body(xf_hbm, mask_hbm, idx_hbm, cov_hbm,
               idx_v, mask_v, idxbuf0, idxbuf1, rows0, rows1, rows_t, cov_v,
               sem0, sem1, semt):
    info = plsc.get_sparse_core_info()
    nc = info.num_cores
    wid = lax.axis_index("s") * nc + lax.axis_index("c")
    b0 = wid * 2

    pltpu.sync_copy(idx_hbm, idx_v)
    pltpu.sync_copy(mask_hbm.at[pl.ds(b0, 2)], mask_v)
    lanes = lax.broadcasted_iota(jnp.int32, (_L,), 0)

    idxbufs = (idxbuf0, idxbuf1)
    rows = (rows0, rows1)
    sems = (sem0, sem1)
    cnts = []

    for j in range(2):
        ibuf = idxbufs[j]
        g_vec = plsc.load_gather(idx_v, [jnp.full((_L,), b0 + j, jnp.int32)])
        base_vec = g_vec * _N

        for k in range(_NCHUNK + 2):
            ibuf[pl.ds(k * _L, _L)] = base_vec

        def compact(c, cnt, ibuf=ibuf, base_vec=base_vec, j=j):
            mv = mask_v[j, pl.ds(c * _L, _L)]
            msk = mv > 0.0
            mi = jnp.where(msk, 1, 0).astype(jnp.int32)
            pos = jnp.cumsum(mi) - 1
            glob = base_vec + c * _L + lanes
            plsc.store_scatter(ibuf, [pos + cnt], glob, mask=msk)
            return cnt + jnp.sum(mi)

        cnt = lax.fori_loop(0, _NCHUNK, compact, jnp.int32(0))
        cnts.append(cnt)
        nchm = jnp.minimum((cnt + _L - 1) // _L, _MCH)

        def fire(t, carry, ibuf=ibuf, rv=rows[j], sem=sems[j]):
            iv = ibuf[pl.ds(t * _L, _L)]
            pltpu.async_copy(xf_hbm.at[iv], rv.at[pl.ds(t * _L, _L)], sem)
            return carry
        lax.fori_loop(0, nchm, fire, jnp.int32(0))

    for j in range(2):
        cnt = cnts[j]
        nchm = jnp.minimum((cnt + _L - 1) // _L, _MCH)
        rv = rows[j]

        def drain(t, carry, ibuf=idxbufs[j], rv=rv, sem=sems[j]):
            iv = ibuf[pl.ds(0, _L)]
            pltpu.make_async_copy(xf_hbm.at[iv],
                                  rv.at[pl.ds(0, _L)], sem).wait()
            return carry
        lax.fori_loop(0, nchm, drain, jnp.int32(0))

        def acc_step(c, acc_c, rv=rv, cnt=cnt):
            out = []
            for s in range(_H // _L):
                seg = acc_c[s]
                for r in range(_L):
                    valid = (c * _L + r) < cnt
                    row = rv[c * _L + r, pl.ds(s * _L, _L)]
                    seg = seg + jnp.where(valid, row, 0.0)
                out.append(seg)
            return tuple(out)

        acc0 = tuple(jnp.zeros((_L,), jnp.float32) for _ in range(_H // _L))
        acc = list(lax.fori_loop(0, nchm, acc_step, acc0))

        nch = (cnt + _L - 1) // _L

        def tail_step(t, acc_t, ibuf=idxbufs[j], cnt=cnt):
            iv = ibuf[pl.ds(t * _L, _L)]
            pltpu.async_copy(xf_hbm.at[iv], rows_t, semt).wait()
            out = []
            for s in range(_H // _L):
                seg = jnp.zeros((_L,), jnp.float32)
                for r in range(_L):
                    valid = (t * _L + r) < cnt
                    row = rows_t[r, pl.ds(s * _L, _L)]
                    seg = seg + jnp.where(valid, row, 0.0)
                out.append(acc_t[s] + seg)
            return tuple(out)

        acc = list(lax.fori_loop(_MCH, nch, tail_step, tuple(acc)))

        denom_vec = jnp.maximum(jnp.full((_L,), cnt.astype(jnp.float32)), 1.0)
        scale = 1.0 / denom_vec
        for s in range(_H // _L):
            cov_v[j, pl.ds(s * _L, _L)] = acc[s] * scale

    pltpu.sync_copy(cov_v, cov_hbm.at[pl.ds(b0, 2)])


def _run_pool(xf_flat, mask_f, idx):
    mesh = plsc.VectorSubcoreMesh(core_axis_name="c", subcore_axis_name="s")
    return pl.kernel(
        _pool_body,
        out_type=jax.ShapeDtypeStruct((_B, _H), jnp.float32),
        mesh=mesh,
        compiler_params=pltpu.CompilerParams(needs_layout_passes=False),
        scratch_types=[
            pltpu.VMEM((_B,), jnp.int32),
            pltpu.VMEM((2, _N), jnp.float32),
            pltpu.VMEM((_N + 2 * _L,), jnp.int32),
            pltpu.VMEM((_N + 2 * _L,), jnp.int32),
            pltpu.VMEM((_MAIN, _H), jnp.float32),
            pltpu.VMEM((_MAIN, _H), jnp.float32),
            pltpu.VMEM((_L, _H), jnp.float32),
            pltpu.VMEM((2, _H), jnp.float32),
            pltpu.SemaphoreType.DMA,
            pltpu.SemaphoreType.DMA,
            pltpu.SemaphoreType.DMA,
        ],
    )(xf_flat, mask_f, idx)


def _head_body(cov_ref, tp_ref, W_tp1_ref, b_tp1_ref, W_tp2_ref, b_tp2_ref,
               W_f1_ref, b_f1_ref, W_f2_ref, b_f2_ref, out_ref):
    t = jnp.dot(tp_ref[...], W_tp1_ref[...],
                preferred_element_type=jnp.float32) + b_tp1_ref[...]
    t = jnp.maximum(t, 0.0)
    t = jnp.dot(t, W_tp2_ref[...],
                preferred_element_type=jnp.float32) + b_tp2_ref[...]
    tp_e = _softmax(t)
    h = (jnp.dot(cov_ref[...], W_f1_ref[:_H],
                 preferred_element_type=jnp.float32)
         + jnp.dot(tp_e, W_f1_ref[_H:],
                   preferred_element_type=jnp.float32)
         + b_f1_ref[...])
    h = jnp.maximum(h, 0.0)
    o = jnp.dot(h, W_f2_ref[...],
                preferred_element_type=jnp.float32) + b_f2_ref[...]
    out_ref[...] = 1.0 / (1.0 + jnp.exp(-o))


def _run_head(cov, test_parameters, W_tp1, b_tp1, W_tp2, b_tp2,
              W_f1, b_f1, W_f2, b_f2):
    return pl.pallas_call(
        _head_body,
        out_shape=jax.ShapeDtypeStruct((_B, 1), jnp.float32),
    )(cov, test_parameters, W_tp1, b_tp1, W_tp2, b_tp2,
      W_f1, b_f1, W_f2, b_f2)


def kernel(test_parameters, graph, coverpoint_mask, graph_xs_all, graph_as_all,
           W_in, b_in, W_gcn, b_gcn, W_tp1, b_tp1, W_tp2, b_tp2,
           W_f1, b_f1, W_f2, b_f2):
    idx = graph[:, 0].astype(jnp.int32)
    mask_f = coverpoint_mask.astype(jnp.float32)

    xf = _run_gcn(graph_xs_all, graph_as_all, W_in, b_in, W_gcn, b_gcn)
    cov = _run_pool(xf.reshape(_G * _N, _H), mask_f, idx)
    out = _run_head(cov, test_parameters, W_tp1, b_tp1, W_tp2, b_tp2,
                    W_f1, b_f1, W_f2, b_f2)
    return out


# speedup vs baseline: 1.9392x; 1.0083x over previous
import jax
import jax.numpy as jnp
from jax import lax
from jax.experimental import pallas as pl
from jax.experimental.pallas import tpu as pltpu
from jax.experimental.pallas import tpu_sc as plsc

_G, _N, _F = 8, 512, 128
_H = 128
_D_TP = 64
_N_MLP = 256
_N_GCN = 4
_B = 64
_GPB = 2
_STEPS = _G // _GPB
_L = 16
_NCHUNK = _N // _L


def _softmax(z):
    z = z - jnp.max(z, axis=-1, keepdims=True)
    e = jnp.exp(z)
    return e / jnp.sum(e, axis=-1, keepdims=True)


def _gcn_body(gx_ref, ga_ref, W_in_ref, b_in_ref, W_gcn_ref, b_gcn_ref,
              tp_ref, W_tp1_ref, b_tp1_ref, W_tp2_ref, b_tp2_ref,
              xf_ref, tpe_ref):
    def bdot(a, b):
        return jnp.dot(a.astype(jnp.bfloat16), b.astype(jnp.bfloat16),
                       preferred_element_type=jnp.float32)

    for j in range(_GPB):
        gx = gx_ref[j]
        ga = ga_ref[j].astype(jnp.bfloat16)
        x = bdot(gx, W_in_ref[...])
        x = jnp.maximum(x + b_in_ref[...], 0.0)
        to_add = x
        for i in range(_N_GCN):
            z = jnp.dot(ga, x.astype(jnp.bfloat16),
                        preferred_element_type=jnp.float32)
            z = bdot(z, W_gcn_ref[i])
            z = z + b_gcn_ref[i]
            if i < _N_GCN - 1:
                x = jnp.maximum(z, 0.0)
            else:
                x = _softmax(z)
        xf_ref[j] = x + to_add

    @pl.when(pl.program_id(0) == _STEPS - 1)
    def _():
        t = jnp.dot(tp_ref[...], W_tp1_ref[...],
                    preferred_element_type=jnp.float32) + b_tp1_ref[...]
        t = jnp.maximum(t, 0.0)
        t = jnp.dot(t, W_tp2_ref[...],
                    preferred_element_type=jnp.float32) + b_tp2_ref[...]
        tpe_ref[...] = _softmax(t)


def _run_gcn(graph_xs_all, graph_as_all, W_in, b_in, W_gcn, b_gcn,
             test_parameters, W_tp1, b_tp1, W_tp2, b_tp2):
    full = lambda shape: pl.BlockSpec(shape, lambda g: (0,) * len(shape))
    return pl.pallas_call(
        _gcn_body,
        grid=(_STEPS,),
        in_specs=[
            pl.BlockSpec((_GPB, _N, _F), lambda g: (g, 0, 0)),
            pl.BlockSpec((_GPB, _N, _N), lambda g: (g, 0, 0)),
            full((_F, _H)), full((_H,)),
            full((_N_GCN, _H, _H)), full((_N_GCN, _H)),
            full((_B, _D_TP)),
            full((_D_TP, _N_MLP)), full((_N_MLP,)),
            full((_N_MLP, _N_MLP)), full((_N_MLP,)),
        ],
        out_specs=[
            pl.BlockSpec((_GPB, _N, _H), lambda g: (g, 0, 0)),
            pl.BlockSpec((_B, _N_MLP), lambda g: (0, 0)),
        ],
        out_shape=[
            jax.ShapeDtypeStruct((_G, _N, _H), jnp.float32),
            jax.ShapeDtypeStruct((_B, _N_MLP), jnp.float32),
        ],
    )(graph_xs_all, graph_as_all, W_in, b_in, W_gcn, b_gcn,
      test_parameters, W_tp1, b_tp1, W_tp2, b_tp2)


_MAIN = 128
_MCH = _MAIN // _L


def _pool_body(xf_hbm, mask_hbm, idx_hbm, cov_hbm,
               idx_v, mask_v, idxbuf0, idxbuf1, rows0, rows1, rows_t, cov_v,
               sem0, sem1, semt):
    info = plsc.get_sparse_core_info()
    nc = info.num_cores
    wid = lax.axis_index("s") * nc + lax.axis_index("c")
    b0 = wid * 2

    pltpu.sync_copy(idx_hbm, idx_v)
    pltpu.sync_copy(mask_hbm.at[pl.ds(b0, 2)], mask_v)
    lanes = lax.broadcasted_iota(jnp.int32, (_L,), 0)

    idxbufs = (idxbuf0, idxbuf1)
    rows = (rows0, rows1)
    sems = (sem0, sem1)
    cnts = []

    for j in range(2):
        ibuf = idxbufs[j]
        g_vec = plsc.load_gather(idx_v, [jnp.full((_L,), b0 + j, jnp.int32)])
        base_vec = g_vec * _N

        for k in range(_NCHUNK + 2):
            ibuf[pl.ds(k * _L, _L)] = base_vec

        def compact(c, cnt, ibuf=ibuf, base_vec=base_vec, j=j):
            mv = mask_v[j, pl.ds(c * _L, _L)]
            msk = mv > 0.0
            mi = jnp.where(msk, 1, 0).astype(jnp.int32)
            pos = jnp.cumsum(mi) - 1
            glob = base_vec + c * _L + lanes
            plsc.store_scatter(ibuf, [pos + cnt], glob, mask=msk)
            return cnt + jnp.sum(mi)

        cnt = lax.fori_loop(0, _NCHUNK, compact, jnp.int32(0))
        cnts.append(cnt)
        nchm = jnp.minimum((cnt + _L - 1) // _L, _MCH)

        def fire(t, carry, ibuf=ibuf, rv=rows[j], sem=sems[j]):
            iv = ibuf[pl.ds(t * _L, _L)]
            pltpu.async_copy(xf_hbm.at[iv], rv.at[pl.ds(t * _L, _L)], sem)
            return carry
        lax.fori_loop(0, nchm, fire, jnp.int32(0))

    for j in range(2):
        cnt = cnts[j]
        nchm = jnp.minimum((cnt + _L - 1) // _L, _MCH)
        rv = rows[j]

        def drain(t, carry, ibuf=idxbufs[j], rv=rv, sem=sems[j]):
            iv = ibuf[pl.ds(0, _L)]
            pltpu.make_async_copy(xf_hbm.at[iv],
                                  rv.at[pl.ds(0, _L)], sem).wait()
            return carry
        lax.fori_loop(0, nchm, drain, jnp.int32(0))

        def acc_step(c, acc_c, rv=rv, cnt=cnt):
            out = []
            for s in range(_H // _L):
                seg = acc_c[s]
                for r in range(_L):
                    valid = (c * _L + r) < cnt
                    row = rv[c * _L + r, pl.ds(s * _L, _L)]
                    seg = seg + jnp.where(valid, row, 0.0)
                out.append(seg)
            return tuple(out)

        acc0 = tuple(jnp.zeros((_L,), jnp.float32) for _ in range(_H // _L))
        acc = list(lax.fori_loop(0, nchm, acc_step, acc0))

        nch = (cnt + _L - 1) // _L

        def tail_step(t, acc_t, ibuf=idxbufs[j], cnt=cnt):
            iv = ibuf[pl.ds(t * _L, _L)]
            pltpu.async_copy(xf_hbm.at[iv], rows_t, semt).wait()
            out = []
            for s in range(_H // _L):
                seg = jnp.zeros((_L,), jnp.float32)
                for r in range(_L):
                    valid = (t * _L + r) < cnt
                    row = rows_t[r, pl.ds(s * _L, _L)]
                    seg = seg + jnp.where(valid, row, 0.0)
                out.append(acc_t[s] + seg)
            return tuple(out)

        acc = list(lax.fori_loop(_MCH, nch, tail_step, tuple(acc)))

        denom_vec = jnp.maximum(jnp.full((_L,), cnt.astype(jnp.float32)), 1.0)
        scale = 1.0 / denom_vec
        for s in range(_H // _L):
            cov_v[j, pl.ds(s * _L, _L)] = acc[s] * scale

    pltpu.sync_copy(cov_v, cov_hbm.at[pl.ds(b0, 2)])


def _run_pool(xf_flat, mask_f, idx):
    mesh = plsc.VectorSubcoreMesh(core_axis_name="c", subcore_axis_name="s")
    return pl.kernel(
        _pool_body,
        out_type=jax.ShapeDtypeStruct((_B, _H), jnp.float32),
        mesh=mesh,
        compiler_params=pltpu.CompilerParams(needs_layout_passes=False),
        scratch_types=[
            pltpu.VMEM((_B,), jnp.int32),
            pltpu.VMEM((2, _N), jnp.float32),
            pltpu.VMEM((_N + 2 * _L,), jnp.int32),
            pltpu.VMEM((_N + 2 * _L,), jnp.int32),
            pltpu.VMEM((_MAIN, _H), jnp.float32),
            pltpu.VMEM((_MAIN, _H), jnp.float32),
            pltpu.VMEM((_L, _H), jnp.float32),
            pltpu.VMEM((2, _H), jnp.float32),
            pltpu.SemaphoreType.DMA,
            pltpu.SemaphoreType.DMA,
            pltpu.SemaphoreType.DMA,
        ],
    )(xf_flat, mask_f, idx)


def _head_body(cov_ref, tpe_ref, W_f1_ref, b_f1_ref, W_f2_ref, b_f2_ref,
               out_ref):
    h = (jnp.dot(cov_ref[...], W_f1_ref[:_H],
                 preferred_element_type=jnp.float32)
         + jnp.dot(tpe_ref[...], W_f1_ref[_H:],
                   preferred_element_type=jnp.float32)
         + b_f1_ref[...])
    h = jnp.maximum(h, 0.0)
    o = jnp.dot(h, W_f2_ref[...],
                preferred_element_type=jnp.float32) + b_f2_ref[...]
    out_ref[...] = 1.0 / (1.0 + jnp.exp(-o))


def _run_head(cov, tp_e, W_f1, b_f1, W_f2, b_f2):
    return pl.pallas_call(
        _head_body,
        out_shape=jax.ShapeDtypeStruct((_B, 1), jnp.float32),
    )(cov, tp_e, W_f1, b_f1, W_f2, b_f2)


def kernel(test_parameters, graph, coverpoint_mask, graph_xs_all, graph_as_all,
           W_in, b_in, W_gcn, b_gcn, W_tp1, b_tp1, W_tp2, b_tp2,
           W_f1, b_f1, W_f2, b_f2):
    idx = graph[:, 0].astype(jnp.int32)
    mask_f = coverpoint_mask.astype(jnp.float32)

    xf, tp_e = _run_gcn(graph_xs_all, graph_as_all, W_in, b_in, W_gcn, b_gcn,
                        test_parameters, W_tp1, b_tp1, W_tp2, b_tp2)
    cov = _run_pool(xf.reshape(_G * _N, _H), mask_f, idx)
    out = _run_head(cov, tp_e, W_f1, b_f1, W_f2, b_f2)
    return out
